# baseline (speedup 1.0000x reference)
import functools

import numpy as np

import jax
import jax.numpy as jnp
from jax import lax
from jax.experimental import pallas as pl
from jax.experimental.pallas import tpu as pltpu
from jax.experimental.pallas import tpu_sc as plsc

NW = 32
LANES = 16


def _depth_weights(depths, depth, dsize):
    depths = jnp.asarray(depths, dtype=jnp.float32)
    depth = jnp.asarray(depth, dtype=jnp.float32)
    iota = jnp.arange(dsize, dtype=jnp.int32)
    ind = jnp.sum((depths <= depth).astype(jnp.int32))
    dind = ind - 1
    dl = jnp.sum(jnp.where(iota == dind, depths, 0.0))
    du = jnp.sum(jnp.where(iota == ind, depths, 0.0))
    depth_r = (depth - dl) / (du - dl)
    dc = dind.astype(jnp.float32) + depth_r
    td = depths - jnp.concatenate([jnp.zeros((1,), jnp.float32), depths[:-1]])
    d1 = jnp.sum(jnp.where(iota == dind, td, 0.0))
    d2 = jnp.sum(jnp.where(iota == dind + 1, td, 0.0))
    d3 = jnp.sum(jnp.where(iota == dind + 2, td, 0.0))

    i0 = dc.astype(jnp.int32)
    s = jnp.clip(dc - i0.astype(jnp.float32), 0.0, 1.0)
    s2_ = s * s
    h0 = (2.0 * s - 3.0) * s2_ + 1.0
    h1 = (3.0 - 2.0 * s) * s2_
    h2 = ((s - 2.0) * s + 1.0) * s
    h3 = (s - 1.0) * s2_

    def rev_tangents(a, b):
        q = a / b
        r = b / a
        inv = 1.0 / (a + b)
        return (-r * inv, (r - q) * inv, q * inv)

    t10, t11, t12 = rev_tangents(d1, d2)
    t20, t21, t22 = rev_tangents(d2, d3)
    s10, s11, s12 = d2 * t10, d2 * t11, d2 * t12
    s20, s21, s22 = d2 * t20, d2 * t21, d2 * t22

    cd = jnp.stack([
        h2 * s10,
        h0 + h2 * s11 + h3 * s20,
        h1 + h2 * s12 + h3 * s21,
        h3 * s22,
    ])
    idp = jnp.clip(i0 - 1 + jnp.arange(4, dtype=jnp.int32), 0, dsize - 1)
    return idp, cd


def _collapse_depth(kn, idp, cd):
    D, Z, Y, X = kn.shape
    BZ = 16

    def body(idp_ref, cd_ref, k0, k1, k2, k3, w_ref):
        w_ref[...] = (cd_ref[0] * k0[0] + cd_ref[1] * k1[0]
                      + cd_ref[2] * k2[0] + cd_ref[3] * k3[0])

    in_specs = [
        pl.BlockSpec((1, BZ, Y, X),
                     functools.partial(lambda i, z, idp_ref, cd_ref: (idp_ref[i], z, 0, 0), i))
        for i in range(4)
    ]
    return pl.pallas_call(
        body,
        grid_spec=pltpu.PrefetchScalarGridSpec(
            num_scalar_prefetch=2,
            grid=(Z // BZ,),
            in_specs=in_specs,
            out_specs=pl.BlockSpec((BZ, Y, X), lambda z, idp_ref, cd_ref: (z, 0, 0)),
        ),
        out_shape=jax.ShapeDtypeStruct((Z, Y, X), jnp.float32),
    )(idp, cd, kn, kn, kn, kn)


def _cr_weights(f):
    w0 = ((-0.5 * f + 1.0) * f - 0.5) * f
    w1 = ((1.5 * f - 2.5) * f) * f + 1.0
    w2 = ((-1.5 * f + 2.0) * f + 0.5) * f
    w3 = ((0.5 * f - 0.5) * f) * f
    return (w0, w1, w2, w3)


def _make_sc_eval(Z, Y, X, N, PT):
    G = PT // LANES
    assert G % 4 == 0 and PT % 8 == 0
    SA = X // 16
    SB = X // 16 - 1
    AROWS = Y * SA
    PROWS = Y * (SA + SB) + Y
    RT = Z * PROWS
    ZPT = Z // LANES

    def body(w_h, idx_h, out_h, t_h,
             idx_v, out_v, pa, pb0, pb1,
             i00, i01, i10, i11, i20, i21, i30, i31,
             r00, r01, r10, r11, r20, r21, r30, r31,
             semA, semP, sem0, sem1, sem2, sem3):
        cid = lax.axis_index("c")
        sid = lax.axis_index("s")
        wid = sid * 2 + cid
        base = wid * PT
        cb = cid * RT

        ib = [(i00, i01), (i10, i11), (i20, i21), (i30, i31)]
        rb = [(r00, r01), (r10, r11), (r20, r21), (r30, r31)]
        sems = [sem0, sem1, sem2, sem3]

        pltpu.sync_copy(idx_h.at[pl.ds(base, PT + 3 * LANES)], idx_v)

        iota = lax.iota(jnp.int32, 16)

        for p in range(ZPT):
            z = sid * ZPT + p
            pltpu.make_async_copy(
                w_h.at[z], t_h.at[pl.ds(cb + z * PROWS, AROWS)], semA).start()
        nstores = 0
        for p in range(ZPT):
            z = sid * ZPT + p
            pltpu.sync_copy(w_h.at[z], pa)
            pb = pb0 if p % 2 == 0 else pb1
            if p >= 2:
                pltpu.make_async_copy(
                    pb, t_h.at[pl.ds(cb, Y * SB)], semP).wait()
                nstores -= 1

            def build(y, c):
                f = y * X + 8
                for s in range(SB):
                    gi = f + 16 * s + iota
                    v = plsc.load_gather(pa, [gi >> 4, gi & 15])
                    pb[y * SB + s, :] = v
                return c

            lax.fori_loop(0, Y, build, 0)
            pltpu.make_async_copy(
                pb, t_h.at[pl.ds(cb + z * PROWS + AROWS, Y * SB)], semP).start()
            nstores += 1
        for _ in range(nstores):
            pltpu.make_async_copy(pb0, t_h.at[pl.ds(cb, Y * SB)], semP).wait()
        for p in range(ZPT):
            z = sid * ZPT + p
            pltpu.make_async_copy(
                w_h.at[z], t_h.at[pl.ds(cb + z * PROWS, AROWS)], semA).wait()
        plsc.subcore_barrier()

        c0 = jnp.zeros((16,), jnp.int32)
        c1 = c0 + 1
        c2 = c0 + 2
        rv = [iota + 16 * t8 for t8 in range(8)]

        def coords(g):
            pvec = iota + g * LANES
            zc = plsc.load_gather(idx_v, [pvec, c0])
            yc = plsc.load_gather(idx_v, [pvec, c1])
            xc = plsc.load_gather(idx_v, [pvec, c2])
            return zc, yc, xc

        def x_layout(xi):
            ix0 = jnp.clip(xi - 1, 0, X - 4)
            use_b = (ix0 & 15) > 12
            e = ix0 - jnp.where(use_b, 8, 0)
            slots = jnp.where(use_b, SB, SA)
            boff = jnp.where(use_b, AROWS, 0)
            return boff + (e >> 4), e & 15, slots

        def fire(g, bi):
            ij0, ij1 = ib[bi]
            zc, yc, xc = coords(g)
            iz0 = jnp.clip(zc.astype(jnp.int32) - 1, 0, Z - 4)
            iy0 = jnp.clip(yc.astype(jnp.int32) - 1, 0, Y - 4)
            blk, _, slots = x_layout(xc.astype(jnp.int32))
            rowbase = cb + iz0 * PROWS + iy0 * slots + blk
            dy1 = slots
            dy2 = dy1 + dy1
            dy3 = dy2 + dy1
            dyo = [c0, dy1, dy2, dy3]
            rz = [rowbase + dz * PROWS for dz in range(4)]
            for t in range(8):
                ij0[pl.ds(t * 16, 16)] = rz[t >> 2] + dyo[t & 3]
            for t in range(8, 16):
                ij1[pl.ds((t - 8) * 16, 16)] = rz[t >> 2] + dyo[t & 3]
            pltpu.make_async_copy(t_h.at[ij0], rb[bi][0], sems[bi]).start()
            pltpu.make_async_copy(t_h.at[ij1], rb[bi][1], sems[bi]).start()

        def drain(bi):
            pltpu.make_async_copy(t_h.at[ib[bi][0]], rb[bi][0], sems[bi]).wait()
            pltpu.make_async_copy(t_h.at[ib[bi][1]], rb[bi][1], sems[bi]).wait()

        def combine(g, bi):
            rj0, rj1 = rb[bi]
            zc, yc, xc = coords(g)
            zi = zc.astype(jnp.int32)
            yi = yc.astype(jnp.int32)
            xi = xc.astype(jnp.int32)
            cz = _cr_weights(zc - zi.astype(jnp.float32))
            cy = _cr_weights(yc - yi.astype(jnp.float32))
            cx = _cr_weights(xc - xi.astype(jnp.float32))
            _, off, _ = x_layout(xi)
            ox = [off + k for k in range(4)]
            acc = [jnp.zeros((16,), jnp.float32) for _ in range(4)]
            for t in range(16):
                czy = cz[t >> 2] * cy[t & 3]
                rj = rj0 if t < 8 else rj1
                for k in range(4):
                    val = plsc.load_gather(rj, [rv[t & 7], ox[k]])
                    acc[k] = acc[k] + val * czy
            out = (acc[0] * cx[0] + acc[1] * cx[1]) + (acc[2] * cx[2] + acc[3] * cx[3])
            out_v[pl.ds(g * LANES, LANES)] = out

        fire(0, 0)
        fire(1, 1)
        fire(2, 2)

        def loop(q, carry):
            g = 4 * q
            for j in range(4):
                fire(g + j + 3, (j + 3) & 3)
                drain(j)
                combine(g + j, j)
            return carry

        lax.fori_loop(0, G // 4, loop, 0)
        for j in range(3):
            drain(j)
        pltpu.sync_copy(out_v, out_h.at[pl.ds(base, PT)])

    mesh = plsc.VectorSubcoreMesh(core_axis_name="c", subcore_axis_name="s")
    return pl.kernel(
        body,
        out_type=(jax.ShapeDtypeStruct((NW * PT,), jnp.float32),
                  jax.ShapeDtypeStruct((2 * RT, 16), jnp.float32)),
        mesh=mesh,
        compiler_params=pltpu.CompilerParams(
            needs_layout_passes=False, use_tc_tiling_on_sc=False),
        scratch_types=(
            [pltpu.VMEM((PT + 3 * LANES, 3), jnp.float32),
             pltpu.VMEM((PT,), jnp.float32),
             pltpu.VMEM((AROWS, 16), jnp.float32),
             pltpu.VMEM((Y * SB, 16), jnp.float32),
             pltpu.VMEM((Y * SB, 16), jnp.float32)]
            + [pltpu.VMEM((128,), jnp.int32)] * 8
            + [pltpu.VMEM((128, 16), jnp.float32)] * 8
            + [pltpu.SemaphoreType.DMA] * 6
        ),
    )


def kernel(idx, knots, depths, depth):
    D, Z, Y, X, C = knots.shape
    N = idx.shape[0]
    kn = knots.reshape(D, Z, Y, X)

    idp, cd = _depth_weights(depths, depth, D)
    W = _collapse_depth(kn, idp, cd)
    w16 = W.reshape(Z, Y * X // 16, 16)

    PT = ((N + NW * 64 - 1) // (NW * 64)) * 64
    out, _ = _make_sc_eval(Z, Y, X, N, PT)(w16, idx)
    return out[:N, None]


# speedup vs baseline: 2.2720x; 2.2720x over previous
import functools

import numpy as np

import jax
import jax.numpy as jnp
from jax import lax
from jax.experimental import pallas as pl
from jax.experimental.pallas import tpu as pltpu
from jax.experimental.pallas import tpu_sc as plsc

NW = 32
LANES = 16
SLOTS = 12


def _depth_weights(depths, depth, dsize):
    depths = jnp.asarray(depths, dtype=jnp.float32)
    depth = jnp.asarray(depth, dtype=jnp.float32)
    iota = jnp.arange(dsize, dtype=jnp.int32)
    ind = jnp.sum((depths <= depth).astype(jnp.int32))
    dind = ind - 1
    dl = jnp.sum(jnp.where(iota == dind, depths, 0.0))
    du = jnp.sum(jnp.where(iota == ind, depths, 0.0))
    depth_r = (depth - dl) / (du - dl)
    dc = dind.astype(jnp.float32) + depth_r
    td = depths - jnp.concatenate([jnp.zeros((1,), jnp.float32), depths[:-1]])
    d1 = jnp.sum(jnp.where(iota == dind, td, 0.0))
    d2 = jnp.sum(jnp.where(iota == dind + 1, td, 0.0))
    d3 = jnp.sum(jnp.where(iota == dind + 2, td, 0.0))

    i0 = dc.astype(jnp.int32)
    s = jnp.clip(dc - i0.astype(jnp.float32), 0.0, 1.0)
    s2_ = s * s
    h0 = (2.0 * s - 3.0) * s2_ + 1.0
    h1 = (3.0 - 2.0 * s) * s2_
    h2 = ((s - 2.0) * s + 1.0) * s
    h3 = (s - 1.0) * s2_

    def rev_tangents(a, b):
        q = a / b
        r = b / a
        inv = 1.0 / (a + b)
        return (-r * inv, (r - q) * inv, q * inv)

    t10, t11, t12 = rev_tangents(d1, d2)
    t20, t21, t22 = rev_tangents(d2, d3)
    s10, s11, s12 = d2 * t10, d2 * t11, d2 * t12
    s20, s21, s22 = d2 * t20, d2 * t21, d2 * t22

    cd = jnp.stack([
        h2 * s10,
        h0 + h2 * s11 + h3 * s20,
        h1 + h2 * s12 + h3 * s21,
        h3 * s22,
    ])
    idp = jnp.clip(i0 - 1 + jnp.arange(4, dtype=jnp.int32), 0, dsize - 1)
    return idp, cd


def _collapse_depth(kn, idp, cd):
    D, Z, Y, X = kn.shape
    BZ = 16

    def body(idp_ref, cd_ref, k0, k1, k2, k3, w_ref):
        w_ref[...] = (cd_ref[0] * k0[0] + cd_ref[1] * k1[0]
                      + cd_ref[2] * k2[0] + cd_ref[3] * k3[0])

    in_specs = [
        pl.BlockSpec((1, BZ, Y, X),
                     functools.partial(lambda i, z, idp_ref, cd_ref: (idp_ref[i], z, 0, 0), i))
        for i in range(4)
    ]
    return pl.pallas_call(
        body,
        grid_spec=pltpu.PrefetchScalarGridSpec(
            num_scalar_prefetch=2,
            grid=(Z // BZ,),
            in_specs=in_specs,
            out_specs=pl.BlockSpec((BZ, Y, X), lambda z, idp_ref, cd_ref: (z, 0, 0)),
        ),
        out_shape=jax.ShapeDtypeStruct((Z, Y, X), jnp.float32),
    )(idp, cd, kn, kn, kn, kn)


def _cr_weights(f):
    w0 = ((-0.5 * f + 1.0) * f - 0.5) * f
    w1 = ((1.5 * f - 2.5) * f) * f + 1.0
    w2 = ((-1.5 * f + 2.0) * f + 0.5) * f
    w3 = ((0.5 * f - 0.5) * f) * f
    return (w0, w1, w2, w3)


def _make_sc_eval(Z, Y, X, PT):
    G = PT // LANES
    assert G % 4 == 0 and PT % 8 == 0
    PTL = PT + 3 * LANES
    LC = [((t >> 2) * Y + (t & 3)) * SLOTS for t in range(16)]

    def body(t_h, zc_h, yc_h, xc_h, out_h,
             zc_v, yc_v, xc_v, out_v,
             i00, i01, i10, i11, i20, i21, i30, i31,
             r00, r01, r10, r11, r20, r21, r30, r31,
             sem0, sem1, sem2, sem3):
        wid = lax.axis_index("s") * 2 + lax.axis_index("c")
        base = wid * PT
        ib = [(i00, i01), (i10, i11), (i20, i21), (i30, i31)]
        rb = [(r00, r01), (r10, r11), (r20, r21), (r30, r31)]
        sems = [sem0, sem1, sem2, sem3]
        pltpu.sync_copy(zc_h.at[pl.ds(base, PTL)], zc_v)
        pltpu.sync_copy(yc_h.at[pl.ds(base, PTL)], yc_v)
        pltpu.sync_copy(xc_h.at[pl.ds(base, PTL)], xc_v)

        iota = lax.iota(jnp.int32, 16)
        rv = [iota + 16 * t8 for t8 in range(8)]

        def x_layout(xi):
            ix0 = jnp.clip(xi - 1, 0, X - 4)
            use_b = (ix0 & 15) > 12
            e = ix0 - jnp.where(use_b, 8, 0)
            sel = jnp.where(use_b, 6, 0)
            return sel + (e >> 4), e & 15

        def fire(g, bi):
            ij0, ij1 = ib[bi]
            st = g * LANES
            zc = zc_v[pl.ds(st, LANES)]
            yc = yc_v[pl.ds(st, LANES)]
            xc = xc_v[pl.ds(st, LANES)]
            iz0 = jnp.clip(zc.astype(jnp.int32) - 1, 0, Z - 4)
            iy0 = jnp.clip(yc.astype(jnp.int32) - 1, 0, Y - 4)
            blk, _ = x_layout(xc.astype(jnp.int32))
            rowbase = (iz0 * Y + iy0) * SLOTS + blk
            for t in range(8):
                ij0[pl.ds(t * 16, 16)] = rowbase + LC[t]
            for t in range(8, 16):
                ij1[pl.ds((t - 8) * 16, 16)] = rowbase + LC[t]
            pltpu.make_async_copy(t_h.at[ij0], rb[bi][0], sems[bi]).start()
            pltpu.make_async_copy(t_h.at[ij1], rb[bi][1], sems[bi]).start()

        def drain(bi):
            pltpu.make_async_copy(t_h.at[ib[bi][0]], rb[bi][0], sems[bi]).wait()
            pltpu.make_async_copy(t_h.at[ib[bi][1]], rb[bi][1], sems[bi]).wait()

        def combine(g, bi):
            rj0, rj1 = rb[bi]
            st = g * LANES
            zc = zc_v[pl.ds(st, LANES)]
            yc = yc_v[pl.ds(st, LANES)]
            xc = xc_v[pl.ds(st, LANES)]
            zi = zc.astype(jnp.int32)
            yi = yc.astype(jnp.int32)
            xi = xc.astype(jnp.int32)
            cz = _cr_weights(zc - zi.astype(jnp.float32))
            cy = _cr_weights(yc - yi.astype(jnp.float32))
            cx = _cr_weights(xc - xi.astype(jnp.float32))
            _, off = x_layout(xi)
            ox = [off + k for k in range(4)]
            acc = [jnp.zeros((16,), jnp.float32) for _ in range(4)]
            for t in range(16):
                czy = cz[t >> 2] * cy[t & 3]
                rj = rj0 if t < 8 else rj1
                for k in range(4):
                    val = plsc.load_gather(rj, [rv[t & 7], ox[k]])
                    acc[k] = acc[k] + val * czy
            out = (acc[0] * cx[0] + acc[1] * cx[1]) + (acc[2] * cx[2] + acc[3] * cx[3])
            out_v[pl.ds(st, LANES)] = out

        fire(0, 0)
        fire(1, 1)
        fire(2, 2)

        def loop(q, carry):
            g = 4 * q
            for j in range(4):
                fire(g + j + 3, (j + 3) & 3)
                drain(j)
                combine(g + j, j)
            return carry

        lax.fori_loop(0, G // 4, loop, 0)
        for j in range(3):
            drain(j)
        pltpu.sync_copy(out_v, out_h.at[pl.ds(base, PT)])

    mesh = plsc.VectorSubcoreMesh(core_axis_name="c", subcore_axis_name="s")
    return pl.kernel(
        body,
        out_type=jax.ShapeDtypeStruct((NW * PT,), jnp.float32),
        mesh=mesh,
        compiler_params=pltpu.CompilerParams(
            needs_layout_passes=False, use_tc_tiling_on_sc=False),
        scratch_types=(
            [pltpu.VMEM((PTL,), jnp.float32)] * 3
            + [pltpu.VMEM((PT,), jnp.float32)]
            + [pltpu.VMEM((128,), jnp.int32)] * 8
            + [pltpu.VMEM((128, 16), jnp.float32)] * 8
            + [pltpu.SemaphoreType.DMA] * 4
        ),
    )


def kernel(idx, knots, depths, depth):
    D, Z, Y, X, C = knots.shape
    N = idx.shape[0]
    kn = knots.reshape(D, Z, Y, X)

    idp, cd = _depth_weights(depths, depth, D)
    W = _collapse_depth(kn, idp, cd)

    a = W.reshape(Z, Y, X // 16, 16)
    b = W[:, :, 8:X - 8].reshape(Z, Y, X // 16 - 1, 16)
    pad = jnp.zeros((Z, Y, SLOTS - (2 * (X // 16) - 1), 16), jnp.float32)
    table = jnp.concatenate([a, b, pad], axis=2).reshape(Z * Y * SLOTS, 16)

    PT = ((N + NW * 64 - 1) // (NW * 64)) * 64
    npc = NW * PT + 3 * LANES
    zc = jnp.pad(idx[:, 0], (0, npc - N), constant_values=2.0)
    yc = jnp.pad(idx[:, 1], (0, npc - N), constant_values=2.0)
    xc = jnp.pad(idx[:, 2], (0, npc - N), constant_values=2.0)

    out = _make_sc_eval(Z, Y, X, PT)(table, zc, yc, xc)
    return out[:N, None]
